# Initial kernel scaffold; baseline (speedup 1.0000x reference)
import jax
import jax.numpy as jnp
from jax.experimental import pallas as pl


def kernel(features, embeddings):
    raise NotImplementedError("write your pallas kernel here")


# speedup vs baseline: 1.5695x; 1.5695x over previous
import jax
import jax.numpy as jnp
from jax import lax
from jax.experimental import pallas as pl
from jax.experimental.pallas import tpu as pltpu

_EPS = 1e-6
_SUBLANE = 8


def _round_up(x: int, m: int) -> int:
    return ((x + m - 1) // m) * m


def _vfe_kernel(f_ref, e_ref, o_ref):
    fb = f_ref[...].astype(jnp.bfloat16)
    acc = jnp.dot(fb, e_ref[...], preferred_element_type=jnp.float32)
    sumsq = jnp.sum(acc * acc, axis=1, keepdims=True)
    inv = jnp.minimum(lax.rsqrt(sumsq), 1.0 / _EPS)
    o_ref[...] = (acc * inv).astype(o_ref.dtype)


def kernel(features: jax.Array, embeddings: jax.Array) -> jax.Array:
    N, K = features.shape
    K2, D = embeddings.shape
    assert K == K2

    e = embeddings.astype(jnp.bfloat16)

    tn = 512
    if N <= tn:
        tn = _round_up(N, _SUBLANE)
    grid = (pl.cdiv(N, tn),)

    f_bytes = 4 * tn * K
    e_bytes = 2 * K * D
    o_bytes = 4 * tn * D
    vmem_limit = int(min(2 * (f_bytes + o_bytes) + 2 * e_bytes + 8 * tn * D,
                         96 * 1024 * 1024))

    return pl.pallas_call(
        _vfe_kernel,
        out_shape=jax.ShapeDtypeStruct((N, D), features.dtype),
        grid=grid,
        in_specs=[
            pl.BlockSpec((tn, K), lambda i: (i, 0)),
            pl.BlockSpec((K, D), lambda i: (0, 0)),
        ],
        out_specs=pl.BlockSpec((tn, D), lambda i: (i, 0)),
        compiler_params=pltpu.CompilerParams(
            dimension_semantics=("parallel",),
            vmem_limit_bytes=vmem_limit,
        ),
        cost_estimate=pl.CostEstimate(
            flops=2 * N * K * D,
            bytes_accessed=4 * N * K + 2 * K * D + 4 * N * D,
            transcendentals=N,
        ),
    )(features, e)


# speedup vs baseline: 1.7565x; 1.1191x over previous
import jax
import jax.numpy as jnp
from jax import lax
from jax.experimental import pallas as pl
from jax.experimental.pallas import tpu as pltpu

_EPS = 1e-6
_SUBLANE = 8


def _round_up(x: int, m: int) -> int:
    return ((x + m - 1) // m) * m


def _vfe_kernel(f_ref, e_ref, o_ref):
    fb = f_ref[...].astype(jnp.bfloat16)
    acc = jnp.dot(fb, e_ref[...], preferred_element_type=jnp.float32)
    sumsq = jnp.sum(acc * acc, axis=1, keepdims=True)
    inv = jnp.minimum(lax.rsqrt(sumsq), 1.0 / _EPS)
    o_ref[...] = (acc * inv).astype(o_ref.dtype)


def kernel(features: jax.Array, embeddings: jax.Array) -> jax.Array:
    N, K = features.shape
    K2, D = embeddings.shape
    assert K == K2

    e = embeddings.astype(jnp.bfloat16)

    tn = 1024
    if N <= tn:
        tn = _round_up(N, _SUBLANE)
    grid = (pl.cdiv(N, tn),)

    f_bytes = 4 * tn * K
    e_bytes = 2 * K * D
    o_bytes = 4 * tn * D
    vmem_limit = int(min(2 * (f_bytes + o_bytes) + 2 * e_bytes + 8 * tn * D,
                         96 * 1024 * 1024))

    return pl.pallas_call(
        _vfe_kernel,
        out_shape=jax.ShapeDtypeStruct((N, D), features.dtype),
        grid=grid,
        in_specs=[
            pl.BlockSpec((tn, K), lambda i: (i, 0)),
            pl.BlockSpec((K, D), lambda i: (0, 0)),
        ],
        out_specs=pl.BlockSpec((tn, D), lambda i: (i, 0)),
        compiler_params=pltpu.CompilerParams(
            dimension_semantics=("parallel",),
            vmem_limit_bytes=vmem_limit,
        ),
        cost_estimate=pl.CostEstimate(
            flops=2 * N * K * D,
            bytes_accessed=4 * N * K + 2 * K * D + 4 * N * D,
            transcendentals=N,
        ),
    )(features, e)


# speedup vs baseline: 1.7721x; 1.0089x over previous
import jax
import jax.numpy as jnp
from jax import lax
from jax.experimental import pallas as pl
from jax.experimental.pallas import tpu as pltpu

_EPS = 1e-6
_SUBLANE = 8


def _round_up(x: int, m: int) -> int:
    return ((x + m - 1) // m) * m


def _vfe_kernel(f_ref, e_ref, o_ref):
    fb = f_ref[...].astype(jnp.bfloat16)
    eb = e_ref[...].astype(jnp.bfloat16)
    acc = jnp.dot(fb, eb, preferred_element_type=jnp.float32)
    sumsq = jnp.sum(acc * acc, axis=1, keepdims=True)
    inv = jnp.minimum(lax.rsqrt(sumsq), 1.0 / _EPS)
    o_ref[...] = (acc * inv).astype(o_ref.dtype)


def kernel(features: jax.Array, embeddings: jax.Array) -> jax.Array:
    N, K = features.shape
    K2, D = embeddings.shape
    assert K == K2

    tn = 1024
    if N <= tn:
        tn = _round_up(N, _SUBLANE)
    grid = (pl.cdiv(N, tn),)

    f_bytes = 4 * tn * K
    e_bytes = 4 * K * D
    o_bytes = 4 * tn * D
    vmem_limit = int(min(2 * (f_bytes + o_bytes) + 2 * e_bytes + 12 * tn * D
                         + 2 * K * D, 96 * 1024 * 1024))

    return pl.pallas_call(
        _vfe_kernel,
        out_shape=jax.ShapeDtypeStruct((N, D), features.dtype),
        grid=grid,
        in_specs=[
            pl.BlockSpec((tn, K), lambda i: (i, 0)),
            pl.BlockSpec((K, D), lambda i: (0, 0)),
        ],
        out_specs=pl.BlockSpec((tn, D), lambda i: (i, 0)),
        compiler_params=pltpu.CompilerParams(
            dimension_semantics=("parallel",),
            vmem_limit_bytes=vmem_limit,
        ),
        cost_estimate=pl.CostEstimate(
            flops=2 * N * K * D,
            bytes_accessed=4 * N * K + 4 * K * D + 4 * N * D,
            transcendentals=N,
        ),
    )(features, embeddings)


# speedup vs baseline: 1.8123x; 1.0227x over previous
import jax
import jax.numpy as jnp
from jax import lax
from jax.experimental import pallas as pl
from jax.experimental.pallas import tpu as pltpu

_EPS = 1e-6
_SUBLANE = 8


def _round_up(x: int, m: int) -> int:
    return ((x + m - 1) // m) * m


def _vfe_kernel(f_ref, e_ref, o_ref):
    fb = f_ref[...].astype(jnp.bfloat16)
    eb = e_ref[...].astype(jnp.bfloat16)
    acc = jnp.dot(fb, eb, preferred_element_type=jnp.float32)
    sumsq = jnp.sum(acc * acc, axis=1, keepdims=True)
    inv = jnp.minimum(lax.rsqrt(sumsq), 1.0 / _EPS)
    o_ref[...] = (acc * inv).astype(o_ref.dtype)


def kernel(features: jax.Array, embeddings: jax.Array) -> jax.Array:
    N, K = features.shape
    K2, D = embeddings.shape
    assert K == K2

    tn = 2048
    if N <= tn:
        tn = _round_up(N, _SUBLANE)
    grid = (pl.cdiv(N, tn),)

    f_bytes = 4 * tn * K
    e_bytes = 4 * K * D
    o_bytes = 4 * tn * D
    vmem_limit = int(min(2 * (f_bytes + o_bytes) + 2 * e_bytes + 12 * tn * D
                         + 2 * K * D, 96 * 1024 * 1024))

    return pl.pallas_call(
        _vfe_kernel,
        out_shape=jax.ShapeDtypeStruct((N, D), features.dtype),
        grid=grid,
        in_specs=[
            pl.BlockSpec((tn, K), lambda i: (i, 0)),
            pl.BlockSpec((K, D), lambda i: (0, 0)),
        ],
        out_specs=pl.BlockSpec((tn, D), lambda i: (i, 0)),
        compiler_params=pltpu.CompilerParams(
            dimension_semantics=("parallel",),
            vmem_limit_bytes=vmem_limit,
        ),
        cost_estimate=pl.CostEstimate(
            flops=2 * N * K * D,
            bytes_accessed=4 * N * K + 4 * K * D + 4 * N * D,
            transcendentals=N,
        ),
    )(features, embeddings)
